# baseline (speedup 1.0000x reference)
import functools

import jax
import jax.numpy as jnp
from jax import lax
from jax.experimental import pallas as pl
from jax.experimental.pallas import tpu as pltpu
from jax.experimental.pallas import tpu_sc as plsc

NUM_ACTIONS = 100000
EMBED_DIM = 32
BATCH = 16384
EPS = 1e-5

NC = 2
NS = 16
L = 16
NW = NC * NS
B_PER_W = BATCH // NW
GATHER_CHUNK = 128
N_CHUNKS = B_PER_W // GATHER_CHUNK
GROUPS = B_PER_W // L


def _rsqrt(x):
    xi = plsc.bitcast(x, jnp.int32)
    y = plsc.bitcast(jnp.int32(0x5F3759DF) - (xi >> 1), jnp.float32)
    half = x * 0.5
    for _ in range(3):
        y = y * (1.5 - half * y * y)
    return y


@functools.partial(
    pl.kernel,
    out_type=jax.ShapeDtypeStruct((BATCH, EMBED_DIM), jnp.float32),
    mesh=plsc.VectorSubcoreMesh(core_axis_name="c", subcore_axis_name="s"),
    compiler_params=pltpu.CompilerParams(
        needs_layout_passes=False, use_tc_tiling_on_sc=False),
    scratch_types=[
        pltpu.VMEM((B_PER_W,), jnp.int32),
        pltpu.VMEM((B_PER_W, EMBED_DIM), jnp.float32),
        pltpu.VMEM((EMBED_DIM,), jnp.float32),
        pltpu.VMEM((EMBED_DIM,), jnp.float32),
        pltpu.SemaphoreType.DMA,
    ],
)
def _sc_embed_ln(idx_hbm, table_hbm, gamma_hbm, beta_hbm, out_hbm,
                 idx_v, rows_v, gamma_v, beta_v, sem):
    wid = lax.axis_index("s") * NC + lax.axis_index("c")
    base = wid * B_PER_W

    pltpu.sync_copy(gamma_hbm, gamma_v)
    pltpu.sync_copy(beta_hbm, beta_v)
    pltpu.sync_copy(idx_hbm.at[pl.ds(base, B_PER_W)], idx_v)

    copies = []
    for c in range(N_CHUNKS):
        sl = pl.ds(c * GATHER_CHUNK, GATHER_CHUNK)
        copies.append(
            pltpu.async_copy(table_hbm.at[idx_v.at[sl]], rows_v.at[sl], sem))
    for cp in copies:
        cp.wait()

    iota16 = lax.iota(jnp.int32, L)
    inv_d = jnp.float32(1.0 / EMBED_DIM)

    def group_body(g, carry):
        ridx = iota16 + g * L
        cols = []
        s = jnp.zeros((L,), jnp.float32)
        for j in range(EMBED_DIM):
            cj = jnp.full((L,), j, jnp.int32)
            v = plsc.load_gather(rows_v, [ridx, cj])
            cols.append(v)
            s = s + v
        mean = s * inv_d
        q = jnp.zeros((L,), jnp.float32)
        for j in range(EMBED_DIM):
            cols[j] = cols[j] - mean
            q = q + cols[j] * cols[j]
        scale = _rsqrt(q * inv_d + jnp.float32(EPS))
        for j in range(EMBED_DIM):
            cj = jnp.full((L,), j, jnp.int32)
            gj = plsc.load_gather(gamma_v, [cj])
            bj = plsc.load_gather(beta_v, [cj])
            o = cols[j] * (scale * gj) + bj
            plsc.store_scatter(rows_v, [ridx, cj], o)
        return carry

    lax.fori_loop(0, GROUPS, group_body, 0)

    pltpu.sync_copy(rows_v, out_hbm.at[pl.ds(base, B_PER_W)])


def kernel(action_indices, table, gamma, beta):
    return _sc_embed_ln(action_indices.astype(jnp.int32), table, gamma, beta)


# speedup vs baseline: 1.2709x; 1.2709x over previous
import functools

import jax
import jax.numpy as jnp
from jax import lax
from jax.experimental import pallas as pl
from jax.experimental.pallas import tpu as pltpu
from jax.experimental.pallas import tpu_sc as plsc

NUM_ACTIONS = 100000
EMBED_DIM = 32
BATCH = 16384
EPS = 1e-5

NC = 2
NS = 16
L = 16
NW = NC * NS
B_PER_W = BATCH // NW
GATHER_CHUNK = 128
N_CHUNKS = B_PER_W // GATHER_CHUNK
GROUPS_PER_CHUNK = GATHER_CHUNK // L

_GATHER_DNUMS = lax.GatherDimensionNumbers(
    offset_dims=(), collapsed_slice_dims=(0,), start_index_map=(0,))


def _lane_broadcast(vec, j):
    sel = jnp.full((L, 1), j, jnp.int32)
    return lax.gather(vec, sel, _GATHER_DNUMS, slice_sizes=(1,),
                      mode=lax.GatherScatterMode.PROMISE_IN_BOUNDS)


def _rsqrt(x):
    xi = plsc.bitcast(x, jnp.int32)
    y = plsc.bitcast(jnp.int32(0x5F3759DF) - (xi >> 1), jnp.float32)
    half = x * 0.5
    y = y * (1.5 - half * y * y)
    y = y * (1.5 - half * y * y)
    return y


@functools.partial(
    pl.kernel,
    out_type=jax.ShapeDtypeStruct((EMBED_DIM, BATCH), jnp.float32),
    mesh=plsc.VectorSubcoreMesh(core_axis_name="c", subcore_axis_name="s"),
    compiler_params=pltpu.CompilerParams(
        needs_layout_passes=False, use_tc_tiling_on_sc=False),
    scratch_types=[
        pltpu.VMEM((B_PER_W,), jnp.int32),
        pltpu.VMEM((B_PER_W, EMBED_DIM), jnp.float32),
        pltpu.VMEM((EMBED_DIM, B_PER_W), jnp.float32),
        pltpu.VMEM((EMBED_DIM,), jnp.float32),
        pltpu.VMEM((EMBED_DIM,), jnp.float32),
        pltpu.SemaphoreType.DMA,
        pltpu.SemaphoreType.DMA,
        pltpu.SemaphoreType.DMA,
        pltpu.SemaphoreType.DMA,
    ],
)
def _sc_embed_ln(idx_hbm, table_hbm, gamma_hbm, beta_hbm, out_hbm,
                 idx_v, rows_v, rows_t, gamma_v, beta_v, s0, s1, s2, s3):
    wid = lax.axis_index("s") * NC + lax.axis_index("c")
    base = wid * B_PER_W

    pltpu.sync_copy(gamma_hbm, gamma_v)
    pltpu.sync_copy(beta_hbm, beta_v)
    pltpu.sync_copy(idx_hbm.at[pl.ds(base, B_PER_W)], idx_v)

    sems = [s0, s1, s2, s3]
    copies = []
    for c in range(N_CHUNKS):
        sl = pl.ds(c * GATHER_CHUNK, GATHER_CHUNK)
        copies.append(
            pltpu.async_copy(table_hbm.at[idx_v.at[sl]], rows_v.at[sl],
                             sems[c]))

    iota16 = lax.iota(jnp.int32, L)
    inv_d = jnp.float32(1.0 / EMBED_DIM)
    g_lo = gamma_v[pl.ds(0, L)]
    g_hi = gamma_v[pl.ds(L, L)]
    b_lo = beta_v[pl.ds(0, L)]
    b_hi = beta_v[pl.ds(L, L)]

    def group_body(g, carry):
        ridx = iota16 + g * L
        cols = []
        acc = [None] * 4
        for j in range(EMBED_DIM):
            cj = jnp.full((L,), j, jnp.int32)
            v = plsc.load_gather(rows_v, [ridx, cj])
            cols.append(v)
            k = j & 3
            acc[k] = v if acc[k] is None else acc[k] + v
        mean = ((acc[0] + acc[1]) + (acc[2] + acc[3])) * inv_d
        qcc = [None] * 4
        for j in range(EMBED_DIM):
            cols[j] = cols[j] - mean
            sq = cols[j] * cols[j]
            k = j & 3
            qcc[k] = sq if qcc[k] is None else qcc[k] + sq
        q = (qcc[0] + qcc[1]) + (qcc[2] + qcc[3])
        scale = _rsqrt(q * inv_d + jnp.float32(EPS))
        for j in range(EMBED_DIM):
            gj = _lane_broadcast(g_lo if j < L else g_hi, j % L)
            bj = _lane_broadcast(b_lo if j < L else b_hi, j % L)
            o = cols[j] * (scale * gj) + bj
            rows_t[j, pl.ds(g * L, L)] = o
        return carry

    for c in range(N_CHUNKS):
        copies[c].wait()
        lax.fori_loop(c * GROUPS_PER_CHUNK, (c + 1) * GROUPS_PER_CHUNK,
                      group_body, 0)

    pltpu.sync_copy(rows_t, out_hbm.at[:, pl.ds(base, B_PER_W)])


def kernel(action_indices, table, gamma, beta):
    out_t = _sc_embed_ln(action_indices.astype(jnp.int32), table, gamma, beta)
    return out_t.T
